# baseline (speedup 1.0000x reference)
import functools

import jax
import jax.numpy as jnp
from jax import lax
from jax.experimental import pallas as pl
from jax.experimental.pallas import tpu as pltpu
from jax.experimental.pallas import tpu_sc as plsc


def kernel(x, branch_idx, weight, bias):
    T, D = x.shape
    idx = branch_idx.astype(jnp.int32)

    info = plsc.get_sparse_core_info()
    NC, NS, L = info.num_cores, info.num_subcores, info.num_lanes
    NW = NC * NS
    tpw = T // NW
    NT = 8
    nblk = tpw // NT
    cpt = D // L

    mesh = plsc.VectorSubcoreMesh(core_axis_name="c", subcore_axis_name="s")

    @functools.partial(
        pl.kernel,
        mesh=mesh,
        out_type=jax.ShapeDtypeStruct((T, D), jnp.float32),
        scratch_types=[
            pltpu.VMEM((NT,), jnp.int32),
            pltpu.VMEM((NT, D), jnp.float32),
            pltpu.VMEM((NT, D), jnp.float32),
            pltpu.VMEM((NT, D), jnp.float32),
            pltpu.VMEM((NT, D), jnp.float32),
            pltpu.SemaphoreType.DMA,
        ],
    )
    def run(x_hbm, idx_hbm, w_hbm, b_hbm, out_hbm, idx_v, x_v, w_v, b_v, o_v, sem):
        wid = lax.axis_index("s") * NC + lax.axis_index("c")
        base = wid * tpw

        def blk(j, carry):
            t0 = base + j * NT
            pltpu.sync_copy(idx_hbm.at[pl.ds(t0, NT)], idx_v)
            pltpu.async_copy(w_hbm.at[idx_v], w_v, sem).wait()
            pltpu.async_copy(b_hbm.at[idx_v], b_v, sem).wait()
            pltpu.sync_copy(x_hbm.at[pl.ds(t0, NT), :], x_v)

            def token(t, c2):
                def chunk(i, c3):
                    sl = pl.ds(i * L, L)
                    o_v[t, sl] = x_v[t, sl] * w_v[t, sl] + b_v[t, sl]
                    return c3

                return lax.fori_loop(0, cpt, chunk, c2)

            lax.fori_loop(0, NT, token, 0)
            pltpu.sync_copy(o_v, out_hbm.at[pl.ds(t0, NT), :])
            return carry

        lax.fori_loop(0, nblk, blk, 0)

    return run(x, idx, weight, bias)
